# Initial kernel scaffold; baseline (speedup 1.0000x reference)
import jax
import jax.numpy as jnp
from jax.experimental import pallas as pl


def kernel(atom_types, coords, edge_index, params):
    raise NotImplementedError("write your pallas kernel here")


# speedup vs baseline: 3.4519x; 3.4519x over previous
import jax
import jax.numpy as jnp
from jax import lax
from jax.experimental import pallas as pl
from jax.experimental.pallas import tpu as pltpu
from jax.experimental.pallas import tpu_sc as plsc

NNODE = 50000
NEDGE = 800000
D = 64
VOCAB = 100

P = 50176
HALF = 25088
SP_ROWS = 25600
ZCH = SP_ROWS // 16
WCH = HALF // 16
ECH = 128
NCHUNK = 392
ETILE = ECH * NCHUNK
EPAD = 16 * ETILE
NC, NS = 2, 16
BB = 1024
NB = P // BB

f32 = jnp.float32
i32 = jnp.int32


def _sc_mesh():
    return plsc.VectorSubcoreMesh(
        core_axis_name="c", subcore_axis_name="s", num_cores=NC, num_subcores=NS
    )


def _agg_body(h_hbm, rowp_hbm, colp_hbm, out_hbm,
              rows_v, row_v, lidx_v, col_v, stage_v, acc_sp, sem):
    c = lax.axis_index("c")
    s = lax.axis_index("s")
    zv = jnp.zeros((16,), f32)

    def zfill(i, carry):
        for jj in range(4):
            stage_v[i, pl.ds(jj * 16, 16)] = zv
        return carry

    lax.fori_loop(0, 50, zfill, 0)

    def zcopy(k, carry):
        pltpu.sync_copy(stage_v, acc_sp.at[pl.ds(s * ZCH + k * 50, 50)])
        return carry

    lax.fori_loop(0, ZCH // 50, zcopy, 0)
    plsc.subcore_barrier()
    half_base = c * HALF

    def chunk(i, carry):
        off = s * ETILE + i * ECH
        pltpu.sync_copy(rowp_hbm.at[pl.ds(off, ECH)], row_v)
        pltpu.sync_copy(colp_hbm.at[pl.ds(off, ECH)], col_v)
        cp = pltpu.async_copy(h_hbm.at[col_v], rows_v, sem)

        def lane(j, c2):
            r = row_v[pl.ds(j * 16, 16)]
            local = r - half_base
            valid = (local >= 0) & (local < HALF)
            trash = HALF + lax.bitwise_and(r, 511)
            lidx_v[pl.ds(j * 16, 16)] = jnp.where(valid, local, trash)
            return c2

        lax.fori_loop(0, ECH // 16, lane, 0)
        cp.wait()
        pltpu.sync_copy(rows_v, acc_sp.at[lidx_v], add=True)
        return carry

    lax.fori_loop(0, NCHUNK, chunk, 0)
    plsc.subcore_barrier()
    woff = s * WCH

    def wcopy(k, carry):
        pltpu.sync_copy(acc_sp.at[pl.ds(woff + k * 49, 49)],
                        stage_v.at[pl.ds(0, 49)])
        pltpu.sync_copy(stage_v.at[pl.ds(0, 49)],
                        out_hbm.at[pl.ds(half_base + woff + k * 49, 49)])
        return carry

    lax.fori_loop(0, WCH // 49, wcopy, 0)


def _segment_sum(h, rowp, colp):
    f = pl.kernel(
        _agg_body,
        out_type=jax.ShapeDtypeStruct((P, D), f32),
        mesh=_sc_mesh(),
        scratch_types=[
            pltpu.VMEM((ECH, D), f32),
            pltpu.VMEM((ECH,), i32),
            pltpu.VMEM((ECH,), i32),
            pltpu.VMEM((ECH,), i32),
            pltpu.VMEM((50, D), f32),
            pltpu.VMEM_SHARED((SP_ROWS, D), f32),
            pltpu.SemaphoreType.DMA,
        ],
        compiler_params=pltpu.CompilerParams(use_tc_tiling_on_sc=False),
    )
    return f(h, rowp, colp)


def _deg_body(rowp_hbm, out_hbm, row_v, lidx_v, ones_v, dstage_v, acc_sp):
    c = lax.axis_index("c")
    s = lax.axis_index("s")
    zv = jnp.zeros((16,), f32)
    ov = jnp.ones((16,), f32)

    def zfill(i, carry):
        dstage_v[pl.ds(i * 16, 16)] = zv
        return carry

    lax.fori_loop(0, ZCH // 16, zfill, 0)

    def ofill(i, carry):
        ones_v[pl.ds(i * 16, 16)] = ov
        return carry

    lax.fori_loop(0, ECH // 16, ofill, 0)
    pltpu.sync_copy(dstage_v, acc_sp.at[pl.ds(s * ZCH, ZCH)])
    plsc.subcore_barrier()
    half_base = c * HALF

    def chunk(i, carry):
        off = s * ETILE + i * ECH
        pltpu.sync_copy(rowp_hbm.at[pl.ds(off, ECH)], row_v)

        def lane(j, c2):
            r = row_v[pl.ds(j * 16, 16)]
            local = r - half_base
            valid = (local >= 0) & (local < HALF)
            trash = HALF + lax.bitwise_and(r, 511)
            lidx_v[pl.ds(j * 16, 16)] = jnp.where(valid, local, trash)
            return c2

        lax.fori_loop(0, ECH // 16, lane, 0)
        pltpu.sync_copy(ones_v, acc_sp.at[lidx_v], add=True)
        return carry

    lax.fori_loop(0, NCHUNK, chunk, 0)
    plsc.subcore_barrier()
    woff = s * WCH
    pltpu.sync_copy(acc_sp.at[pl.ds(woff, WCH)], dstage_v.at[pl.ds(0, WCH)])
    pltpu.sync_copy(dstage_v.at[pl.ds(0, WCH)],
                    out_hbm.at[pl.ds(half_base + woff, WCH)])


def _degree(rowp):
    f = pl.kernel(
        _deg_body,
        out_type=jax.ShapeDtypeStruct((P,), f32),
        mesh=_sc_mesh(),
        scratch_types=[
            pltpu.VMEM((ECH,), i32),
            pltpu.VMEM((ECH,), i32),
            pltpu.VMEM((ECH,), f32),
            pltpu.VMEM((ZCH,), f32),
            pltpu.VMEM_SHARED((SP_ROWS,), f32),
        ],
    )
    return f(rowp)


def _emb_kernel(at_ref, emb_ref, out_ref):
    a = at_ref[...]
    oh = (a == lax.broadcasted_iota(i32, (1, 128), 1)).astype(f32)
    out_ref[...] = jnp.dot(oh, emb_ref[...], preferred_element_type=f32)


def _embed(at2, embp):
    return pl.pallas_call(
        _emb_kernel,
        grid=(NB,),
        in_specs=[
            pl.BlockSpec((BB, 1), lambda i: (i, 0)),
            pl.BlockSpec((128, D), lambda i: (0, 0)),
        ],
        out_specs=pl.BlockSpec((BB, D), lambda i: (i, 0)),
        out_shape=jax.ShapeDtypeStruct((P, D), f32),
    )(at2, embp)


def _layer_kernel(h_ref, agg_ref, deg_ref, w1_ref, b1_ref, w2_ref, b2_ref,
                  out_ref):
    dg = jnp.maximum(deg_ref[...], 1.0)
    a = agg_ref[...] / dg
    z = jnp.maximum(jnp.dot(a, w1_ref[...], preferred_element_type=f32)
                    + b1_ref[...], 0.0)
    out_ref[...] = (h_ref[...]
                    + jnp.dot(z, w2_ref[...], preferred_element_type=f32)
                    + b2_ref[...])


def _layer(h, agg, deg2, w1, b1, w2, b2):
    return pl.pallas_call(
        _layer_kernel,
        grid=(NB,),
        in_specs=[
            pl.BlockSpec((BB, D), lambda i: (i, 0)),
            pl.BlockSpec((BB, D), lambda i: (i, 0)),
            pl.BlockSpec((BB, 1), lambda i: (i, 0)),
            pl.BlockSpec((D, D), lambda i: (0, 0)),
            pl.BlockSpec((1, D), lambda i: (0, 0)),
            pl.BlockSpec((D, D), lambda i: (0, 0)),
            pl.BlockSpec((1, D), lambda i: (0, 0)),
        ],
        out_specs=pl.BlockSpec((BB, D), lambda i: (i, 0)),
        out_shape=jax.ShapeDtypeStruct((P, D), f32),
    )(h, agg, deg2, w1, b1, w2, b2)


def _head_kernel(h_ref, rw1_ref, rb1_ref, rw2t_ref, rb2_ref, out_ref, acc_ref):
    i = pl.program_id(0)

    @pl.when(i == 0)
    def _():
        acc_ref[...] = jnp.zeros_like(acc_ref)

    base = i * BB
    valid = (lax.broadcasted_iota(i32, (BB, 1), 0) + base) < NNODE
    hm = jnp.where(valid, h_ref[...], 0.0)
    acc_ref[...] += jnp.sum(hm, axis=0, keepdims=True)

    @pl.when(i == NB - 1)
    def _():
        g = acc_ref[...] * (1.0 / NNODE)
        r = jnp.maximum(jnp.dot(g, rw1_ref[...], preferred_element_type=f32)
                        + rb1_ref[...], 0.0)
        o = jnp.sum(r * rw2t_ref[...], axis=1, keepdims=True) + rb2_ref[...]
        out_ref[...] = jax.nn.sigmoid(o)


def _head(h, rw1, rb1, rw2t, rb2):
    return pl.pallas_call(
        _head_kernel,
        grid=(NB,),
        in_specs=[
            pl.BlockSpec((BB, D), lambda i: (i, 0)),
            pl.BlockSpec((D, 32), lambda i: (0, 0)),
            pl.BlockSpec((1, 32), lambda i: (0, 0)),
            pl.BlockSpec((1, 32), lambda i: (0, 0)),
            pl.BlockSpec((1, 1), lambda i: (0, 0)),
        ],
        out_specs=pl.BlockSpec((1, 1), lambda i: (0, 0)),
        out_shape=jax.ShapeDtypeStruct((1, 1), f32),
        scratch_shapes=[pltpu.VMEM((1, D), f32)],
        compiler_params=pltpu.CompilerParams(
            dimension_semantics=("arbitrary",)),
    )(h, rw1, rb1, rw2t, rb2)


def kernel(atom_types, coords, edge_index, params):
    del coords
    at2 = jnp.pad(atom_types.astype(i32), (0, P - NNODE)).reshape(P, 1)
    row = edge_index[0].astype(i32)
    col = edge_index[1].astype(i32)
    rowp = jnp.pad(row, (0, EPAD - NEDGE), constant_values=1 << 20)
    colp = jnp.pad(col, (0, EPAD - NEDGE))
    embp = jnp.pad(params["embedding"].astype(f32), ((0, 128 - VOCAB), (0, 0)))

    h = _embed(at2, embp)
    deg2 = _degree(rowp).reshape(P, 1)
    for layer in params["layers"]:
        agg = _segment_sum(h, rowp, colp)
        h = _layer(h, agg, deg2,
                   layer["W1"].astype(f32), layer["b1"].reshape(1, D),
                   layer["W2"].astype(f32), layer["b2"].reshape(1, D))
    out = _head(h, params["rW1"].astype(f32),
                params["rb1"].reshape(1, 32),
                params["rW2"].reshape(1, 32),
                params["rb2"].reshape(1, 1))
    return out.reshape(1)


# speedup vs baseline: 5.2458x; 1.5197x over previous
import jax
import jax.numpy as jnp
from jax import lax
from jax.experimental import pallas as pl
from jax.experimental.pallas import tpu as pltpu
from jax.experimental.pallas import tpu_sc as plsc

NNODE = 50000
NEDGE = 800000
D = 64
VOCAB = 100

P = 50176
HALF = 25088
SP_ROWS = 25600
ZCH = SP_ROWS // 16
WCH = HALF // 16
ECH = 128
SUP = 1024
NSUP = 49
NCHUNK = 392
ETILE = SUP * NSUP
EPAD = 16 * ETILE
NC, NS = 2, 16
BB = 1024
NB = P // BB

f32 = jnp.float32
i32 = jnp.int32


def _sc_mesh():
    return plsc.VectorSubcoreMesh(
        core_axis_name="c", subcore_axis_name="s", num_cores=NC, num_subcores=NS
    )


def _agg_body(h_hbm, rowp_hbm, colp2_hbm, out_hbm,
              rows_buf0, rows_buf1, row_v, lidx2_v, col2_v, stage_v, acc_sp,
              gsem, ssem):
    c = lax.axis_index("c")
    s = lax.axis_index("s")
    rows_bufs = [rows_buf0, rows_buf1]
    nbuf = 2
    jn = SUP // ECH
    zv = jnp.zeros((16,), f32)

    def zfill(i, carry):
        for jj in range(4):
            stage_v[i, pl.ds(jj * 16, 16)] = zv
        return carry

    lax.fori_loop(0, 50, zfill, 0)

    def zcopy(k, carry):
        pltpu.sync_copy(stage_v, acc_sp.at[pl.ds(s * ZCH + k * 50, 50)])
        return carry

    lax.fori_loop(0, ZCH // 50, zcopy, 0)
    plsc.subcore_barrier()
    half_base = c * HALF

    def superchunk(i, carry):
        off = s * ETILE + i * SUP
        pltpu.sync_copy(rowp_hbm.at[pl.ds(off, SUP)], row_v)
        pltpu.sync_copy(colp2_hbm.at[pl.ds(s * (ETILE // ECH) + i * jn, jn)],
                        col2_v)

        for jr in range(jn):
            def lane(j, c2):
                r = row_v[pl.ds(jr * ECH + j * 16, 16)]
                local = r - half_base
                valid = (local >= 0) & (local < HALF)
                trash = HALF + lax.bitwise_and(r, 511)
                lidx2_v[jr, pl.ds(j * 16, 16)] = jnp.where(valid, local,
                                                           trash)
                return c2

            lax.fori_loop(0, ECH // 16, lane, 0)

        gathers = [None] * jn
        scatters = [None] * jn
        gathers[0] = pltpu.async_copy(
            h_hbm.at[col2_v.at[0]], rows_bufs[0], gsem)
        for j in range(jn):
            gathers[j].wait()
            scatters[j] = pltpu.async_copy(
                rows_bufs[j % nbuf], acc_sp.at[lidx2_v.at[j]], ssem, add=True)
            if j + 1 < jn:
                if j + 1 >= nbuf:
                    scatters[j + 1 - nbuf].wait()
                gathers[j + 1] = pltpu.async_copy(
                    h_hbm.at[col2_v.at[j + 1]], rows_bufs[(j + 1) % nbuf],
                    gsem)
        for j in range(jn - nbuf + 1, jn):
            scatters[j].wait()
        return carry

    lax.fori_loop(0, NSUP, superchunk, 0)
    plsc.subcore_barrier()
    woff = s * WCH

    def wcopy(k, carry):
        pltpu.sync_copy(acc_sp.at[pl.ds(woff + k * 49, 49)],
                        stage_v.at[pl.ds(0, 49)])
        pltpu.sync_copy(stage_v.at[pl.ds(0, 49)],
                        out_hbm.at[pl.ds(half_base + woff + k * 49, 49)])
        return carry

    lax.fori_loop(0, WCH // 49, wcopy, 0)


def _segment_sum(h, rowp, colp2):
    f = pl.kernel(
        _agg_body,
        out_type=jax.ShapeDtypeStruct((P, D), f32),
        mesh=_sc_mesh(),
        scratch_types=[
            pltpu.VMEM((ECH, D), f32),
            pltpu.VMEM((ECH, D), f32),
            pltpu.VMEM((SUP,), i32),
            pltpu.VMEM((SUP // ECH, ECH), i32),
            pltpu.VMEM((SUP // ECH, ECH), i32),
            pltpu.VMEM((50, D), f32),
            pltpu.VMEM_SHARED((SP_ROWS, D), f32),
            pltpu.SemaphoreType.DMA,
            pltpu.SemaphoreType.DMA,
        ],
        compiler_params=pltpu.CompilerParams(use_tc_tiling_on_sc=False),
    )
    return f(h, rowp, colp2)


def _deg_body(rowp_hbm, out_hbm, row_v, lidx_v, ones_v, dstage_v, acc_sp):
    c = lax.axis_index("c")
    s = lax.axis_index("s")
    zv = jnp.zeros((16,), f32)
    ov = jnp.ones((16,), f32)

    def zfill(i, carry):
        dstage_v[pl.ds(i * 16, 16)] = zv
        return carry

    lax.fori_loop(0, ZCH // 16, zfill, 0)

    def ofill(i, carry):
        ones_v[pl.ds(i * 16, 16)] = ov
        return carry

    lax.fori_loop(0, ECH // 16, ofill, 0)
    pltpu.sync_copy(dstage_v, acc_sp.at[pl.ds(s * ZCH, ZCH)])
    plsc.subcore_barrier()
    half_base = c * HALF

    def chunk(i, carry):
        off = s * ETILE + i * ECH
        pltpu.sync_copy(rowp_hbm.at[pl.ds(off, ECH)], row_v)

        def lane(j, c2):
            r = row_v[pl.ds(j * 16, 16)]
            local = r - half_base
            valid = (local >= 0) & (local < HALF)
            trash = HALF + lax.bitwise_and(r, 511)
            lidx_v[pl.ds(j * 16, 16)] = jnp.where(valid, local, trash)
            return c2

        lax.fori_loop(0, ECH // 16, lane, 0)
        pltpu.sync_copy(ones_v, acc_sp.at[lidx_v], add=True)
        return carry

    lax.fori_loop(0, NCHUNK, chunk, 0)
    plsc.subcore_barrier()
    woff = s * WCH
    pltpu.sync_copy(acc_sp.at[pl.ds(woff, WCH)], dstage_v.at[pl.ds(0, WCH)])
    pltpu.sync_copy(dstage_v.at[pl.ds(0, WCH)],
                    out_hbm.at[pl.ds(half_base + woff, WCH)])


def _degree(rowp):
    f = pl.kernel(
        _deg_body,
        out_type=jax.ShapeDtypeStruct((P,), f32),
        mesh=_sc_mesh(),
        scratch_types=[
            pltpu.VMEM((ECH,), i32),
            pltpu.VMEM((ECH,), i32),
            pltpu.VMEM((ECH,), f32),
            pltpu.VMEM((ZCH,), f32),
            pltpu.VMEM_SHARED((SP_ROWS,), f32),
        ],
    )
    return f(rowp)


def _emb_kernel(at_ref, emb_ref, out_ref):
    a = at_ref[...]
    oh = (a == lax.broadcasted_iota(i32, (1, 128), 1)).astype(f32)
    out_ref[...] = jnp.dot(oh, emb_ref[...], preferred_element_type=f32)


def _embed(at2, embp):
    return pl.pallas_call(
        _emb_kernel,
        grid=(NB,),
        in_specs=[
            pl.BlockSpec((BB, 1), lambda i: (i, 0)),
            pl.BlockSpec((128, D), lambda i: (0, 0)),
        ],
        out_specs=pl.BlockSpec((BB, D), lambda i: (i, 0)),
        out_shape=jax.ShapeDtypeStruct((P, D), f32),
    )(at2, embp)


def _layer_kernel(h_ref, agg_ref, deg_ref, w1_ref, b1_ref, w2_ref, b2_ref,
                  out_ref):
    dg = jnp.maximum(deg_ref[...], 1.0)
    a = agg_ref[...] / dg
    z = jnp.maximum(jnp.dot(a, w1_ref[...], preferred_element_type=f32)
                    + b1_ref[...], 0.0)
    out_ref[...] = (h_ref[...]
                    + jnp.dot(z, w2_ref[...], preferred_element_type=f32)
                    + b2_ref[...])


def _layer(h, agg, deg2, w1, b1, w2, b2):
    return pl.pallas_call(
        _layer_kernel,
        grid=(NB,),
        in_specs=[
            pl.BlockSpec((BB, D), lambda i: (i, 0)),
            pl.BlockSpec((BB, D), lambda i: (i, 0)),
            pl.BlockSpec((BB, 1), lambda i: (i, 0)),
            pl.BlockSpec((D, D), lambda i: (0, 0)),
            pl.BlockSpec((1, D), lambda i: (0, 0)),
            pl.BlockSpec((D, D), lambda i: (0, 0)),
            pl.BlockSpec((1, D), lambda i: (0, 0)),
        ],
        out_specs=pl.BlockSpec((BB, D), lambda i: (i, 0)),
        out_shape=jax.ShapeDtypeStruct((P, D), f32),
    )(h, agg, deg2, w1, b1, w2, b2)


def _head_kernel(h_ref, rw1_ref, rb1_ref, rw2t_ref, rb2_ref, out_ref, acc_ref):
    i = pl.program_id(0)

    @pl.when(i == 0)
    def _():
        acc_ref[...] = jnp.zeros_like(acc_ref)

    base = i * BB
    valid = (lax.broadcasted_iota(i32, (BB, 1), 0) + base) < NNODE
    hm = jnp.where(valid, h_ref[...], 0.0)
    acc_ref[...] += jnp.sum(hm, axis=0, keepdims=True)

    @pl.when(i == NB - 1)
    def _():
        g = acc_ref[...] * (1.0 / NNODE)
        r = jnp.maximum(jnp.dot(g, rw1_ref[...], preferred_element_type=f32)
                        + rb1_ref[...], 0.0)
        o = jnp.sum(r * rw2t_ref[...], axis=1, keepdims=True) + rb2_ref[...]
        out_ref[...] = jax.nn.sigmoid(o)


def _head(h, rw1, rb1, rw2t, rb2):
    return pl.pallas_call(
        _head_kernel,
        grid=(NB,),
        in_specs=[
            pl.BlockSpec((BB, D), lambda i: (i, 0)),
            pl.BlockSpec((D, 32), lambda i: (0, 0)),
            pl.BlockSpec((1, 32), lambda i: (0, 0)),
            pl.BlockSpec((1, 32), lambda i: (0, 0)),
            pl.BlockSpec((1, 1), lambda i: (0, 0)),
        ],
        out_specs=pl.BlockSpec((1, 1), lambda i: (0, 0)),
        out_shape=jax.ShapeDtypeStruct((1, 1), f32),
        scratch_shapes=[pltpu.VMEM((1, D), f32)],
        compiler_params=pltpu.CompilerParams(
            dimension_semantics=("arbitrary",)),
    )(h, rw1, rb1, rw2t, rb2)


def kernel(atom_types, coords, edge_index, params):
    del coords
    at2 = jnp.pad(atom_types.astype(i32), (0, P - NNODE)).reshape(P, 1)
    row = edge_index[0].astype(i32)
    col = edge_index[1].astype(i32)
    rowp = jnp.pad(row, (0, EPAD - NEDGE), constant_values=1 << 20)
    colp = jnp.pad(col, (0, EPAD - NEDGE))
    embp = jnp.pad(params["embedding"].astype(f32), ((0, 128 - VOCAB), (0, 0)))

    colp2 = colp.reshape(EPAD // ECH, ECH)
    h = _embed(at2, embp)
    deg2 = _degree(rowp).reshape(P, 1)
    for layer in params["layers"]:
        agg = _segment_sum(h, rowp, colp2)
        h = _layer(h, agg, deg2,
                   layer["W1"].astype(f32), layer["b1"].reshape(1, D),
                   layer["W2"].astype(f32), layer["b2"].reshape(1, D))
    out = _head(h, params["rW1"].astype(f32),
                params["rb1"].reshape(1, 32),
                params["rW2"].reshape(1, 32),
                params["rb2"].reshape(1, 1))
    return out.reshape(1)


# speedup vs baseline: 7.1842x; 1.3695x over previous
import jax
import jax.numpy as jnp
from jax import lax
from jax.experimental import pallas as pl
from jax.experimental.pallas import tpu as pltpu
from jax.experimental.pallas import tpu_sc as plsc

NNODE = 50000
NEDGE = 800000
D = 64
VOCAB = 100

P = 50176
HALF = 25088
SP_ROWS = 25344
TRMASK = 255
ZCH = SP_ROWS // 16
WCH = HALF // 16
ECH = 128
SUP = 1024
NSUP = 49
NCHUNK = 392
ETILE = SUP * NSUP
EPAD = 16 * ETILE
NC, NS = 2, 16
BB = 1024
NB = P // BB

f32 = jnp.float32
i32 = jnp.int32


def _sc_mesh():
    return plsc.VectorSubcoreMesh(
        core_axis_name="c", subcore_axis_name="s", num_cores=NC, num_subcores=NS
    )


def _agg_body(h_hbm, rowp_hbm, colp2_hbm, out_hbm,
              rows_buf0, rows_buf1, rows_buf2, row_v, lidx2_v, col2_v,
              acc_sp, gsem, ssem):
    c = lax.axis_index("c")
    s = lax.axis_index("s")
    rows_bufs = [rows_buf0, rows_buf1, rows_buf2]
    nbuf = 3
    jn = SUP // ECH
    zv = jnp.zeros((16,), f32)

    def zfill(i, carry):
        for jj in range(4):
            rows_buf0[i, pl.ds(jj * 16, 16)] = zv
        return carry

    lax.fori_loop(0, 48, zfill, 0)

    def zcopy(k, carry):
        pltpu.sync_copy(rows_buf0.at[pl.ds(0, 48)],
                        acc_sp.at[pl.ds(s * ZCH + k * 48, 48)])
        return carry

    lax.fori_loop(0, ZCH // 48, zcopy, 0)
    plsc.subcore_barrier()
    half_base = c * HALF

    def superchunk(i, carry):
        off = s * ETILE + i * SUP
        pltpu.sync_copy(colp2_hbm.at[pl.ds(s * (ETILE // ECH) + i * jn, jn)],
                        col2_v)
        gathers = [None] * jn
        scatters = [None] * jn
        gathers[0] = pltpu.async_copy(h_hbm.at[col2_v.at[0]], rows_bufs[0],
                                      gsem)
        gathers[1] = pltpu.async_copy(h_hbm.at[col2_v.at[1]], rows_bufs[1],
                                      gsem)
        pltpu.sync_copy(rowp_hbm.at[pl.ds(off, SUP)], row_v)

        for jr in range(jn):
            def lane(j, c2):
                r = row_v[pl.ds(jr * ECH + j * 16, 16)]
                local = r - half_base
                valid = (local >= 0) & (local < HALF)
                trash = HALF + lax.bitwise_and(r, TRMASK)
                lidx2_v[jr, pl.ds(j * 16, 16)] = jnp.where(valid, local,
                                                           trash)
                return c2

            lax.fori_loop(0, ECH // 16, lane, 0)

        for j in range(jn):
            gathers[j].wait()
            scatters[j] = pltpu.async_copy(
                rows_bufs[j % nbuf], acc_sp.at[lidx2_v.at[j]], ssem, add=True)
            if j + 2 < jn:
                if j >= 1:
                    scatters[j - 1].wait()
                gathers[j + 2] = pltpu.async_copy(
                    h_hbm.at[col2_v.at[j + 2]], rows_bufs[(j + 2) % nbuf],
                    gsem)
        for j in range(jn - 3, jn):
            scatters[j].wait()
        return carry

    lax.fori_loop(0, NSUP, superchunk, 0)
    plsc.subcore_barrier()
    woff = s * WCH

    def wcopy(k, carry):
        pltpu.sync_copy(acc_sp.at[pl.ds(woff + k * 56, 56)],
                        rows_buf0.at[pl.ds(0, 56)])
        pltpu.sync_copy(rows_buf0.at[pl.ds(0, 56)],
                        out_hbm.at[pl.ds(half_base + woff + k * 56, 56)])
        return carry

    lax.fori_loop(0, WCH // 56, wcopy, 0)


def _segment_sum(h, rowp, colp2):
    f = pl.kernel(
        _agg_body,
        out_type=jax.ShapeDtypeStruct((P, D), f32),
        mesh=_sc_mesh(),
        scratch_types=[
            pltpu.VMEM((ECH, D), f32),
            pltpu.VMEM((ECH, D), f32),
            pltpu.VMEM((ECH, D), f32),
            pltpu.VMEM((SUP,), i32),
            pltpu.VMEM((SUP // ECH, ECH), i32),
            pltpu.VMEM((SUP // ECH, ECH), i32),
            pltpu.VMEM_SHARED((SP_ROWS, D), f32),
            pltpu.SemaphoreType.DMA,
            pltpu.SemaphoreType.DMA,
        ],
        compiler_params=pltpu.CompilerParams(use_tc_tiling_on_sc=False),
    )
    return f(h, rowp, colp2)


def _deg_body(rowp_hbm, out_hbm, row_v, lidx2_v, ones_v, dstage_v, acc_sp,
              ssem):
    c = lax.axis_index("c")
    s = lax.axis_index("s")
    jn = SUP // ECH
    zv = jnp.zeros((16,), f32)
    ov = jnp.ones((16,), f32)

    def zfill(i, carry):
        dstage_v[pl.ds(i * 16, 16)] = zv
        return carry

    lax.fori_loop(0, ZCH // 16, zfill, 0)

    def ofill(i, carry):
        ones_v[pl.ds(i * 16, 16)] = ov
        return carry

    lax.fori_loop(0, ECH // 16, ofill, 0)
    pltpu.sync_copy(dstage_v, acc_sp.at[pl.ds(s * ZCH, ZCH)])
    plsc.subcore_barrier()
    half_base = c * HALF

    def superchunk(i, carry):
        off = s * ETILE + i * SUP
        pltpu.sync_copy(rowp_hbm.at[pl.ds(off, SUP)], row_v)

        def lane(k, c2):
            r = row_v[pl.ds(k * 16, 16)]
            local = r - half_base
            valid = (local >= 0) & (local < HALF)
            trash = HALF + lax.bitwise_and(r, TRMASK)
            lidx2_v[k // 8, pl.ds((k % 8) * 16, 16)] = jnp.where(
                valid, local, trash)
            return c2

        lax.fori_loop(0, SUP // 16, lane, 0)
        scatters = [
            pltpu.async_copy(ones_v, acc_sp.at[lidx2_v.at[j]], ssem,
                             add=True)
            for j in range(jn)
        ]
        for sc in scatters:
            sc.wait()
        return carry

    lax.fori_loop(0, NSUP, superchunk, 0)
    plsc.subcore_barrier()
    woff = s * WCH
    pltpu.sync_copy(acc_sp.at[pl.ds(woff, WCH)], dstage_v.at[pl.ds(0, WCH)])
    pltpu.sync_copy(dstage_v.at[pl.ds(0, WCH)],
                    out_hbm.at[pl.ds(half_base + woff, WCH)])


def _degree(rowp):
    f = pl.kernel(
        _deg_body,
        out_type=jax.ShapeDtypeStruct((P,), f32),
        mesh=_sc_mesh(),
        scratch_types=[
            pltpu.VMEM((SUP,), i32),
            pltpu.VMEM((SUP // ECH, ECH), i32),
            pltpu.VMEM((ECH,), f32),
            pltpu.VMEM((ZCH,), f32),
            pltpu.VMEM_SHARED((SP_ROWS,), f32),
            pltpu.SemaphoreType.DMA,
        ],
    )
    return f(rowp)


def _emb_kernel(at_ref, emb_ref, out_ref):
    a = at_ref[...]
    oh = (a == lax.broadcasted_iota(i32, (1, 128), 1)).astype(f32)
    out_ref[...] = jnp.dot(oh, emb_ref[...], preferred_element_type=f32)


def _embed(at2, embp):
    return pl.pallas_call(
        _emb_kernel,
        grid=(NB,),
        in_specs=[
            pl.BlockSpec((BB, 1), lambda i: (i, 0)),
            pl.BlockSpec((128, D), lambda i: (0, 0)),
        ],
        out_specs=pl.BlockSpec((BB, D), lambda i: (i, 0)),
        out_shape=jax.ShapeDtypeStruct((P, D), f32),
    )(at2, embp)


def _layer_kernel(h_ref, agg_ref, deg_ref, w1_ref, b1_ref, w2_ref, b2_ref,
                  out_ref):
    dg = jnp.maximum(deg_ref[...], 1.0)
    a = agg_ref[...] / dg
    z = jnp.maximum(jnp.dot(a, w1_ref[...], preferred_element_type=f32)
                    + b1_ref[...], 0.0)
    out_ref[...] = (h_ref[...]
                    + jnp.dot(z, w2_ref[...], preferred_element_type=f32)
                    + b2_ref[...])


def _layer(h, agg, deg2, w1, b1, w2, b2):
    return pl.pallas_call(
        _layer_kernel,
        grid=(NB,),
        in_specs=[
            pl.BlockSpec((BB, D), lambda i: (i, 0)),
            pl.BlockSpec((BB, D), lambda i: (i, 0)),
            pl.BlockSpec((BB, 1), lambda i: (i, 0)),
            pl.BlockSpec((D, D), lambda i: (0, 0)),
            pl.BlockSpec((1, D), lambda i: (0, 0)),
            pl.BlockSpec((D, D), lambda i: (0, 0)),
            pl.BlockSpec((1, D), lambda i: (0, 0)),
        ],
        out_specs=pl.BlockSpec((BB, D), lambda i: (i, 0)),
        out_shape=jax.ShapeDtypeStruct((P, D), f32),
    )(h, agg, deg2, w1, b1, w2, b2)


def _head_kernel(h_ref, rw1_ref, rb1_ref, rw2t_ref, rb2_ref, out_ref, acc_ref):
    i = pl.program_id(0)

    @pl.when(i == 0)
    def _():
        acc_ref[...] = jnp.zeros_like(acc_ref)

    base = i * BB
    valid = (lax.broadcasted_iota(i32, (BB, 1), 0) + base) < NNODE
    hm = jnp.where(valid, h_ref[...], 0.0)
    acc_ref[...] += jnp.sum(hm, axis=0, keepdims=True)

    @pl.when(i == NB - 1)
    def _():
        g = acc_ref[...] * (1.0 / NNODE)
        r = jnp.maximum(jnp.dot(g, rw1_ref[...], preferred_element_type=f32)
                        + rb1_ref[...], 0.0)
        o = jnp.sum(r * rw2t_ref[...], axis=1, keepdims=True) + rb2_ref[...]
        out_ref[...] = jax.nn.sigmoid(o)


def _head(h, rw1, rb1, rw2t, rb2):
    return pl.pallas_call(
        _head_kernel,
        grid=(NB,),
        in_specs=[
            pl.BlockSpec((BB, D), lambda i: (i, 0)),
            pl.BlockSpec((D, 32), lambda i: (0, 0)),
            pl.BlockSpec((1, 32), lambda i: (0, 0)),
            pl.BlockSpec((1, 32), lambda i: (0, 0)),
            pl.BlockSpec((1, 1), lambda i: (0, 0)),
        ],
        out_specs=pl.BlockSpec((1, 1), lambda i: (0, 0)),
        out_shape=jax.ShapeDtypeStruct((1, 1), f32),
        scratch_shapes=[pltpu.VMEM((1, D), f32)],
        compiler_params=pltpu.CompilerParams(
            dimension_semantics=("arbitrary",)),
    )(h, rw1, rb1, rw2t, rb2)


def kernel(atom_types, coords, edge_index, params):
    del coords
    at2 = jnp.pad(atom_types.astype(i32), (0, P - NNODE)).reshape(P, 1)
    row = edge_index[0].astype(i32)
    col = edge_index[1].astype(i32)
    rowp = jnp.pad(row, (0, EPAD - NEDGE), constant_values=1 << 20)
    colp = jnp.pad(col, (0, EPAD - NEDGE))
    embp = jnp.pad(params["embedding"].astype(f32), ((0, 128 - VOCAB), (0, 0)))

    colp2 = colp.reshape(EPAD // ECH, ECH)
    h = _embed(at2, embp)
    deg2 = _degree(rowp).reshape(P, 1)
    for layer in params["layers"]:
        agg = _segment_sum(h, rowp, colp2)
        h = _layer(h, agg, deg2,
                   layer["W1"].astype(f32), layer["b1"].reshape(1, D),
                   layer["W2"].astype(f32), layer["b2"].reshape(1, D))
    out = _head(h, params["rW1"].astype(f32),
                params["rb1"].reshape(1, 32),
                params["rW2"].reshape(1, 32),
                params["rb2"].reshape(1, 1))
    return out.reshape(1)


# speedup vs baseline: 8.4127x; 1.1710x over previous
import jax
import jax.numpy as jnp
from jax import lax
from jax.experimental import pallas as pl
from jax.experimental.pallas import tpu as pltpu
from jax.experimental.pallas import tpu_sc as plsc

NNODE = 50000
NEDGE = 800000
D = 64
VOCAB = 100

P = 50176
HALF = 25088
SP_ROWS = 25344
TRMASK = 255
ZCH = SP_ROWS // 16
WCH = HALF // 16
ECH = 128
SUP = 1024
NSUP = 49
NCHUNK = 392
ETILE = SUP * NSUP
EPAD = 16 * ETILE
NC, NS = 2, 16
BB = 1024
NB = P // BB

f32 = jnp.float32
i32 = jnp.int32


def _sc_mesh():
    return plsc.VectorSubcoreMesh(
        core_axis_name="c", subcore_axis_name="s", num_cores=NC, num_subcores=NS
    )


def _agg_body(h_hbm, rowp_hbm, colp2_hbm, out_hbm,
              rows_buf0, rows_buf1, rows_buf2, row_v, lidx2_v, col2_v,
              acc_sp, gsem, ssem):
    c = lax.axis_index("c")
    s = lax.axis_index("s")
    rows_bufs = [rows_buf0, rows_buf1, rows_buf2]
    nbuf = 3
    jn = SUP // ECH
    zv = jnp.zeros((16,), f32)

    def zfill(i, carry):
        for jj in range(4):
            rows_buf0[i, pl.ds(jj * 16, 16)] = zv
        return carry

    lax.fori_loop(0, 48, zfill, 0)

    def zcopy(k, carry):
        pltpu.sync_copy(rows_buf0.at[pl.ds(0, 48)],
                        acc_sp.at[pl.ds(s * ZCH + k * 48, 48)])
        return carry

    lax.fori_loop(0, ZCH // 48, zcopy, 0)
    plsc.subcore_barrier()
    half_base = c * HALF

    def superchunk(i, carry):
        off = s * ETILE + i * SUP
        pltpu.sync_copy(colp2_hbm.at[pl.ds(s * (ETILE // ECH) + i * jn, jn)],
                        col2_v)
        gathers = [None] * jn
        scatters = [None] * jn
        gathers[0] = pltpu.async_copy(h_hbm.at[col2_v.at[0]], rows_bufs[0],
                                      gsem)
        gathers[1] = pltpu.async_copy(h_hbm.at[col2_v.at[1]], rows_bufs[1],
                                      gsem)
        pltpu.sync_copy(rowp_hbm.at[pl.ds(off, SUP)], row_v)

        for jr in range(jn):
            def lane(j, c2):
                r = row_v[pl.ds(jr * ECH + j * 16, 16)]
                local = r - half_base
                valid = (local >= 0) & (local < HALF)
                trash = HALF + lax.bitwise_and(r, TRMASK)
                lidx2_v[jr, pl.ds(j * 16, 16)] = jnp.where(valid, local,
                                                           trash)
                return c2

            lax.fori_loop(0, ECH // 16, lane, 0)

        for j in range(jn):
            gathers[j].wait()
            scatters[j] = pltpu.async_copy(
                rows_bufs[j % nbuf], acc_sp.at[lidx2_v.at[j]], ssem, add=True)
            if j + 2 < jn:
                if j >= 1:
                    scatters[j - 1].wait()
                gathers[j + 2] = pltpu.async_copy(
                    h_hbm.at[col2_v.at[j + 2]], rows_bufs[(j + 2) % nbuf],
                    gsem)
        for j in range(jn - 3, jn):
            scatters[j].wait()
        return carry

    lax.fori_loop(0, NSUP, superchunk, 0)
    plsc.subcore_barrier()
    woff = s * WCH

    def wcopy(k, carry):
        pltpu.sync_copy(acc_sp.at[pl.ds(woff + k * 56, 56)],
                        rows_buf0.at[pl.ds(0, 56)])
        pltpu.sync_copy(rows_buf0.at[pl.ds(0, 56)],
                        out_hbm.at[pl.ds(half_base + woff + k * 56, 56)])
        return carry

    lax.fori_loop(0, WCH // 56, wcopy, 0)


def _segment_sum(h, rowp, colp2):
    f = pl.kernel(
        _agg_body,
        out_type=jax.ShapeDtypeStruct((P, D), f32),
        mesh=_sc_mesh(),
        scratch_types=[
            pltpu.VMEM((ECH, D), f32),
            pltpu.VMEM((ECH, D), f32),
            pltpu.VMEM((ECH, D), f32),
            pltpu.VMEM((SUP,), i32),
            pltpu.VMEM((SUP // ECH, ECH), i32),
            pltpu.VMEM((SUP // ECH, ECH), i32),
            pltpu.VMEM_SHARED((SP_ROWS, D), f32),
            pltpu.SemaphoreType.DMA,
            pltpu.SemaphoreType.DMA,
        ],
        compiler_params=pltpu.CompilerParams(use_tc_tiling_on_sc=False),
    )
    return f(h, rowp, colp2)


def _deg_body(rowp_hbm, out_hbm, row_v, lidx2_v, ones_v, dstage_v, acc_sp,
              ssem):
    c = lax.axis_index("c")
    s = lax.axis_index("s")
    jn = SUP // ECH
    zv = jnp.zeros((16,), f32)
    ov = jnp.ones((16,), f32)

    def zfill(i, carry):
        dstage_v[pl.ds(i * 16, 16)] = zv
        return carry

    lax.fori_loop(0, ZCH // 16, zfill, 0)

    def ofill(i, carry):
        ones_v[pl.ds(i * 16, 16)] = ov
        return carry

    lax.fori_loop(0, ECH // 16, ofill, 0)
    pltpu.sync_copy(dstage_v, acc_sp.at[pl.ds(s * ZCH, ZCH)])
    plsc.subcore_barrier()
    half_base = c * HALF

    def superchunk(i, carry):
        off = s * ETILE + i * SUP
        pltpu.sync_copy(rowp_hbm.at[pl.ds(off, SUP)], row_v)

        def lane(k, c2):
            r = row_v[pl.ds(k * 16, 16)]
            local = r - half_base
            valid = (local >= 0) & (local < HALF)
            trash = HALF + lax.bitwise_and(r, TRMASK)
            lidx2_v[k // 8, pl.ds((k % 8) * 16, 16)] = jnp.where(
                valid, local, trash)
            return c2

        lax.fori_loop(0, SUP // 16, lane, 0)
        scatters = [
            pltpu.async_copy(ones_v, acc_sp.at[lidx2_v.at[j]], ssem,
                             add=True)
            for j in range(jn)
        ]
        for sc in scatters:
            sc.wait()
        return carry

    lax.fori_loop(0, NSUP, superchunk, 0)
    plsc.subcore_barrier()
    woff = s * WCH
    pltpu.sync_copy(acc_sp.at[pl.ds(woff, WCH)], dstage_v.at[pl.ds(0, WCH)])
    pltpu.sync_copy(dstage_v.at[pl.ds(0, WCH)],
                    out_hbm.at[pl.ds(half_base + woff, WCH)])


def _degree(rowp):
    f = pl.kernel(
        _deg_body,
        out_type=jax.ShapeDtypeStruct((P,), f32),
        mesh=_sc_mesh(),
        scratch_types=[
            pltpu.VMEM((SUP,), i32),
            pltpu.VMEM((SUP // ECH, ECH), i32),
            pltpu.VMEM((ECH,), f32),
            pltpu.VMEM((ZCH,), f32),
            pltpu.VMEM_SHARED((SP_ROWS,), f32),
            pltpu.SemaphoreType.DMA,
        ],
    )
    return f(rowp)


PH = P // 2
BBH = BB // 2


def _emb_kernel(at_ref, embl_ref, embr_ref, out_ref):
    a = at_ref[...]
    lanes = lax.broadcasted_iota(i32, (1, 128), 1)
    ohe = (a[:, 0:1] == lanes).astype(f32)
    oho = (a[:, 1:2] == lanes).astype(f32)
    out_ref[...] = (
        jnp.dot(ohe, embl_ref[...], preferred_element_type=f32)
        + jnp.dot(oho, embr_ref[...], preferred_element_type=f32))


def _embed(at2, embl, embr):
    return pl.pallas_call(
        _emb_kernel,
        grid=(NB,),
        in_specs=[
            pl.BlockSpec((BBH, 2), lambda i: (i, 0)),
            pl.BlockSpec((128, 128), lambda i: (0, 0)),
            pl.BlockSpec((128, 128), lambda i: (0, 0)),
        ],
        out_specs=pl.BlockSpec((BBH, 128), lambda i: (i, 0)),
        out_shape=jax.ShapeDtypeStruct((PH, 128), f32),
    )(at2, embl, embr)


def _layer_update(h_ref, agg_ref, deg_ref, w1_ref, b1_ref, w2_ref, b2_ref):
    dg = jnp.maximum(deg_ref[...], 1.0)
    dgw = jnp.concatenate(
        [jnp.broadcast_to(dg[:, 0:1], (BBH, D)),
         jnp.broadcast_to(dg[:, 1:2], (BBH, D))], axis=1)
    a = agg_ref[...] / dgw
    z = jnp.maximum(jnp.dot(a, w1_ref[...], preferred_element_type=f32)
                    + b1_ref[...], 0.0)
    return (h_ref[...]
            + jnp.dot(z, w2_ref[...], preferred_element_type=f32)
            + b2_ref[...])


def _layer_kernel(h_ref, agg_ref, deg_ref, w1_ref, b1_ref, w2_ref, b2_ref,
                  out_ref):
    out_ref[...] = _layer_update(h_ref, agg_ref, deg_ref, w1_ref, b1_ref,
                                 w2_ref, b2_ref)


_LAYER_SPECS = [
    pl.BlockSpec((BBH, 128), lambda i: (i, 0)),
    pl.BlockSpec((BBH, 128), lambda i: (i, 0)),
    pl.BlockSpec((BBH, 2), lambda i: (i, 0)),
    pl.BlockSpec((128, 128), lambda i: (0, 0)),
    pl.BlockSpec((1, 128), lambda i: (0, 0)),
    pl.BlockSpec((128, 128), lambda i: (0, 0)),
    pl.BlockSpec((1, 128), lambda i: (0, 0)),
]


def _layer(h, agg, degp, w1b, b1b, w2b, b2b):
    return pl.pallas_call(
        _layer_kernel,
        grid=(NB,),
        in_specs=_LAYER_SPECS,
        out_specs=pl.BlockSpec((BBH, 128), lambda i: (i, 0)),
        out_shape=jax.ShapeDtypeStruct((PH, 128), f32),
    )(h, agg, degp, w1b, b1b, w2b, b2b)


def _layer_head_kernel(h_ref, agg_ref, deg_ref, w1_ref, b1_ref, w2_ref,
                       b2_ref, rw1_ref, rb1_ref, rw2t_ref, rb2_ref,
                       out_ref, acc_ref):
    i = pl.program_id(0)

    @pl.when(i == 0)
    def _():
        acc_ref[...] = jnp.zeros_like(acc_ref)

    hn = _layer_update(h_ref, agg_ref, deg_ref, w1_ref, b1_ref, w2_ref,
                       b2_ref)
    base = i * BBH
    valid = (lax.broadcasted_iota(i32, (BBH, 1), 0) + base) < (NNODE // 2)
    hm = jnp.where(valid, hn, 0.0)
    acc_ref[...] += jnp.sum(hm, axis=0, keepdims=True)

    @pl.when(i == NB - 1)
    def _():
        g = acc_ref[...] * (1.0 / NNODE)
        r = jnp.maximum(jnp.dot(g, rw1_ref[...], preferred_element_type=f32)
                        + rb1_ref[...], 0.0)
        o = jnp.sum(r * rw2t_ref[...], axis=1, keepdims=True) + rb2_ref[...]
        out_ref[...] = jax.nn.sigmoid(o)


def _layer_head(h, agg, degp, w1b, b1b, w2b, b2b, rw1s, rb1, rw2t, rb2):
    return pl.pallas_call(
        _layer_head_kernel,
        grid=(NB,),
        in_specs=_LAYER_SPECS + [
            pl.BlockSpec((128, 32), lambda i: (0, 0)),
            pl.BlockSpec((1, 32), lambda i: (0, 0)),
            pl.BlockSpec((1, 32), lambda i: (0, 0)),
            pl.BlockSpec((1, 1), lambda i: (0, 0)),
        ],
        out_specs=pl.BlockSpec((1, 1), lambda i: (0, 0)),
        out_shape=jax.ShapeDtypeStruct((1, 1), f32),
        scratch_shapes=[pltpu.VMEM((1, 128), f32)],
        compiler_params=pltpu.CompilerParams(
            dimension_semantics=("arbitrary",)),
    )(h, agg, degp, w1b, b1b, w2b, b2b, rw1s, rb1, rw2t, rb2)


def _blockdiag(w):
    z = jnp.zeros((128, 128), f32)
    z = z.at[:D, :D].set(w)
    return z.at[D:, D:].set(w)


def kernel(atom_types, coords, edge_index, params):
    del coords
    at2 = jnp.pad(atom_types.astype(i32), (0, P - NNODE)).reshape(PH, 2)
    row = edge_index[0].astype(i32)
    col = edge_index[1].astype(i32)
    rowp = jnp.pad(row, (0, EPAD - NEDGE), constant_values=1 << 20)
    colp = jnp.pad(col, (0, EPAD - NEDGE))
    embp = jnp.pad(params["embedding"].astype(f32), ((0, 128 - VOCAB), (0, 0)))
    embl = jnp.pad(embp, ((0, 0), (0, D)))
    embr = jnp.pad(embp, ((0, 0), (D, 0)))

    colp2 = colp.reshape(EPAD // ECH, ECH)
    h = _embed(at2, embl, embr)
    degp = _degree(rowp).reshape(PH, 2)
    lw = [
        (_blockdiag(layer["W1"].astype(f32)),
         jnp.tile(layer["b1"].astype(f32), 2).reshape(1, 128),
         _blockdiag(layer["W2"].astype(f32)),
         jnp.tile(layer["b2"].astype(f32), 2).reshape(1, 128))
        for layer in params["layers"]
    ]
    for li in range(2):
        agg = _segment_sum(h.reshape(P, D), rowp, colp2).reshape(PH, 128)
        h = _layer(h, agg, degp, *lw[li])
    agg = _segment_sum(h.reshape(P, D), rowp, colp2).reshape(PH, 128)
    rw1s = jnp.concatenate([params["rW1"].astype(f32),
                            params["rW1"].astype(f32)], axis=0)
    out = _layer_head(h, agg, degp, *lw[2],
                      rw1s, params["rb1"].reshape(1, 32),
                      params["rW2"].reshape(1, 32),
                      params["rb2"].reshape(1, 1))
    return out.reshape(1)


# speedup vs baseline: 8.8207x; 1.0485x over previous
import jax
import jax.numpy as jnp
from jax import lax
from jax.experimental import pallas as pl
from jax.experimental.pallas import tpu as pltpu
from jax.experimental.pallas import tpu_sc as plsc

NNODE = 50000
NEDGE = 800000
D = 64
VOCAB = 100

P = 50176
HALF = 25088
SP_ROWS = 25280
TRMASK = 127
ZCH = SP_ROWS // 16
WCH = HALF // 16
SP_ROWS_D = 25344
ZCH_D = SP_ROWS_D // 16
ECH = 128
SUP = 1024
NSUP = 49
NCHUNK = 392
ETILE = SUP * NSUP
EPAD = 16 * ETILE
NC, NS = 2, 16
BB = 1024
NB = P // BB

f32 = jnp.float32
i32 = jnp.int32


def _sc_mesh():
    return plsc.VectorSubcoreMesh(
        core_axis_name="c", subcore_axis_name="s", num_cores=NC, num_subcores=NS
    )


def _agg_body(h_hbm, rowp_hbm, colp2_hbm, out_hbm,
              rows_buf0, rows_buf1, rows_buf2, row_a, row_b, lidx2_v,
              col2_a, col2_b, acc_sp, gsem, ssem, csem):
    c = lax.axis_index("c")
    s = lax.axis_index("s")
    rows_bufs = [rows_buf0, rows_buf1, rows_buf2]
    nbuf = 3
    jn = SUP // ECH
    zv = jnp.zeros((16,), f32)

    def zfill(i, carry):
        for jj in range(4):
            rows_buf0[i, pl.ds(jj * 16, 16)] = zv
        return carry

    lax.fori_loop(0, 48, zfill, 0)

    def zcopy(k, carry):
        pltpu.sync_copy(rows_buf0.at[pl.ds(0, 48)],
                        acc_sp.at[pl.ds(s * ZCH + k * 48, 48)])
        return carry

    lax.fori_loop(0, ZCH // 48, zcopy, 0)
    pltpu.sync_copy(rows_buf0.at[pl.ds(0, ZCH - 48 * (ZCH // 48))],
                    acc_sp.at[pl.ds(s * ZCH + 48 * (ZCH // 48),
                                    ZCH - 48 * (ZCH // 48))])
    plsc.subcore_barrier()
    half_base = c * HALF

    def _idx_copies(i, rv, cv):
        off = s * ETILE + i * SUP
        return (
            pltpu.make_async_copy(
                colp2_hbm.at[pl.ds(s * (ETILE // ECH) + i * jn, jn)], cv,
                csem),
            pltpu.make_async_copy(rowp_hbm.at[pl.ds(off, SUP)], rv, csem),
        )

    def superchunk(i, row_v, col2_v, pre):
        for d in _idx_copies(i, row_v, col2_v):
            d.wait()
        if pre is not None:
            for d in _idx_copies(*pre):
                d.start()
        gathers = [None] * jn
        scatters = [None] * jn
        gathers[0] = pltpu.async_copy(h_hbm.at[col2_v.at[0]], rows_bufs[0],
                                      gsem)
        gathers[1] = pltpu.async_copy(h_hbm.at[col2_v.at[1]], rows_bufs[1],
                                      gsem)

        for jr in range(jn):
            def lane(j, c2):
                r = row_v[pl.ds(jr * ECH + j * 16, 16)]
                local = r - half_base
                valid = (local >= 0) & (local < HALF)
                trash = HALF + lax.bitwise_and(r, TRMASK)
                lidx2_v[jr, pl.ds(j * 16, 16)] = jnp.where(valid, local,
                                                           trash)
                return c2

            lax.fori_loop(0, ECH // 16, lane, 0)

        for j in range(jn):
            gathers[j].wait()
            scatters[j] = pltpu.async_copy(
                rows_bufs[j % nbuf], acc_sp.at[lidx2_v.at[j]], ssem, add=True)
            if j + 2 < jn:
                if j >= 1:
                    scatters[j - 1].wait()
                gathers[j + 2] = pltpu.async_copy(
                    h_hbm.at[col2_v.at[j + 2]], rows_bufs[(j + 2) % nbuf],
                    gsem)
        for j in range(jn - 3, jn):
            scatters[j].wait()

    for d in _idx_copies(0, row_a, col2_a):
        d.start()

    def pair(p, carry):
        i = p * 2
        superchunk(i, row_a, col2_a, (i + 1, row_b, col2_b))
        superchunk(i + 1, row_b, col2_b, (i + 2, row_a, col2_a))
        return carry

    lax.fori_loop(0, NSUP // 2, pair, 0)
    superchunk(NSUP - 1, row_a, col2_a, None)
    plsc.subcore_barrier()
    woff = s * WCH

    def wcopy(k, carry):
        pltpu.sync_copy(acc_sp.at[pl.ds(woff + k * 56, 56)],
                        rows_buf0.at[pl.ds(0, 56)])
        pltpu.sync_copy(rows_buf0.at[pl.ds(0, 56)],
                        out_hbm.at[pl.ds(half_base + woff + k * 56, 56)])
        return carry

    lax.fori_loop(0, WCH // 56, wcopy, 0)


def _segment_sum(h, rowp, colp2):
    f = pl.kernel(
        _agg_body,
        out_type=jax.ShapeDtypeStruct((P, D), f32),
        mesh=_sc_mesh(),
        scratch_types=[
            pltpu.VMEM((ECH, D), f32),
            pltpu.VMEM((ECH, D), f32),
            pltpu.VMEM((ECH, D), f32),
            pltpu.VMEM((SUP,), i32),
            pltpu.VMEM((SUP,), i32),
            pltpu.VMEM((SUP // ECH, ECH), i32),
            pltpu.VMEM((SUP // ECH, ECH), i32),
            pltpu.VMEM((SUP // ECH, ECH), i32),
            pltpu.VMEM_SHARED((SP_ROWS, D), f32),
            pltpu.SemaphoreType.DMA,
            pltpu.SemaphoreType.DMA,
            pltpu.SemaphoreType.DMA,
        ],
        compiler_params=pltpu.CompilerParams(use_tc_tiling_on_sc=False),
    )
    return f(h, rowp, colp2)


def _deg_body(rowp_hbm, out_hbm, row_v, lidx2_v, ones_v, dstage_v, acc_sp,
              ssem):
    c = lax.axis_index("c")
    s = lax.axis_index("s")
    jn = SUP // ECH
    zv = jnp.zeros((16,), f32)
    ov = jnp.ones((16,), f32)

    def zfill(i, carry):
        dstage_v[pl.ds(i * 16, 16)] = zv
        return carry

    lax.fori_loop(0, ZCH_D // 16, zfill, 0)

    def ofill(i, carry):
        ones_v[pl.ds(i * 16, 16)] = ov
        return carry

    lax.fori_loop(0, ECH // 16, ofill, 0)
    pltpu.sync_copy(dstage_v, acc_sp.at[pl.ds(s * ZCH_D, ZCH_D)])
    plsc.subcore_barrier()
    half_base = c * HALF

    def superchunk(i, carry):
        off = s * ETILE + i * SUP
        pltpu.sync_copy(rowp_hbm.at[pl.ds(off, SUP)], row_v)

        def lane(k, c2):
            r = row_v[pl.ds(k * 16, 16)]
            local = r - half_base
            valid = (local >= 0) & (local < HALF)
            trash = HALF + lax.bitwise_and(r, 255)
            lidx2_v[k // 8, pl.ds((k % 8) * 16, 16)] = jnp.where(
                valid, local, trash)
            return c2

        lax.fori_loop(0, SUP // 16, lane, 0)
        scatters = [
            pltpu.async_copy(ones_v, acc_sp.at[lidx2_v.at[j]], ssem,
                             add=True)
            for j in range(jn)
        ]
        for sc in scatters:
            sc.wait()
        return carry

    lax.fori_loop(0, NSUP, superchunk, 0)
    plsc.subcore_barrier()
    woff = s * WCH
    pltpu.sync_copy(acc_sp.at[pl.ds(woff, WCH)], dstage_v.at[pl.ds(0, WCH)])
    pltpu.sync_copy(dstage_v.at[pl.ds(0, WCH)],
                    out_hbm.at[pl.ds(half_base + woff, WCH)])


def _degree(rowp):
    f = pl.kernel(
        _deg_body,
        out_type=jax.ShapeDtypeStruct((P,), f32),
        mesh=_sc_mesh(),
        scratch_types=[
            pltpu.VMEM((SUP,), i32),
            pltpu.VMEM((SUP // ECH, ECH), i32),
            pltpu.VMEM((ECH,), f32),
            pltpu.VMEM((ZCH_D,), f32),
            pltpu.VMEM_SHARED((SP_ROWS_D,), f32),
            pltpu.SemaphoreType.DMA,
        ],
    )
    return f(rowp)


PH = P // 2
BBH = BB // 2


def _emb_kernel(at_ref, embl_ref, embr_ref, out_ref):
    a = at_ref[...]
    lanes = lax.broadcasted_iota(i32, (1, 128), 1)
    ohe = (a[:, 0:1] == lanes).astype(f32)
    oho = (a[:, 1:2] == lanes).astype(f32)
    out_ref[...] = (
        jnp.dot(ohe, embl_ref[...], preferred_element_type=f32)
        + jnp.dot(oho, embr_ref[...], preferred_element_type=f32))


def _embed(at2, embl, embr):
    return pl.pallas_call(
        _emb_kernel,
        grid=(NB,),
        in_specs=[
            pl.BlockSpec((BBH, 2), lambda i: (i, 0)),
            pl.BlockSpec((128, 128), lambda i: (0, 0)),
            pl.BlockSpec((128, 128), lambda i: (0, 0)),
        ],
        out_specs=pl.BlockSpec((BBH, 128), lambda i: (i, 0)),
        out_shape=jax.ShapeDtypeStruct((PH, 128), f32),
    )(at2, embl, embr)


def _layer_update(h_ref, agg_ref, deg_ref, w1_ref, b1_ref, w2_ref, b2_ref):
    dg = jnp.maximum(deg_ref[...], 1.0)
    dgw = jnp.concatenate(
        [jnp.broadcast_to(dg[:, 0:1], (BBH, D)),
         jnp.broadcast_to(dg[:, 1:2], (BBH, D))], axis=1)
    a = agg_ref[...] / dgw
    z = jnp.maximum(jnp.dot(a, w1_ref[...], preferred_element_type=f32)
                    + b1_ref[...], 0.0)
    return (h_ref[...]
            + jnp.dot(z, w2_ref[...], preferred_element_type=f32)
            + b2_ref[...])


def _layer_kernel(h_ref, agg_ref, deg_ref, w1_ref, b1_ref, w2_ref, b2_ref,
                  out_ref):
    out_ref[...] = _layer_update(h_ref, agg_ref, deg_ref, w1_ref, b1_ref,
                                 w2_ref, b2_ref)


_LAYER_SPECS = [
    pl.BlockSpec((BBH, 128), lambda i: (i, 0)),
    pl.BlockSpec((BBH, 128), lambda i: (i, 0)),
    pl.BlockSpec((BBH, 2), lambda i: (i, 0)),
    pl.BlockSpec((128, 128), lambda i: (0, 0)),
    pl.BlockSpec((1, 128), lambda i: (0, 0)),
    pl.BlockSpec((128, 128), lambda i: (0, 0)),
    pl.BlockSpec((1, 128), lambda i: (0, 0)),
]


def _layer(h, agg, degp, w1b, b1b, w2b, b2b):
    return pl.pallas_call(
        _layer_kernel,
        grid=(NB,),
        in_specs=_LAYER_SPECS,
        out_specs=pl.BlockSpec((BBH, 128), lambda i: (i, 0)),
        out_shape=jax.ShapeDtypeStruct((PH, 128), f32),
    )(h, agg, degp, w1b, b1b, w2b, b2b)


def _layer_head_kernel(h_ref, agg_ref, deg_ref, w1_ref, b1_ref, w2_ref,
                       b2_ref, rw1_ref, rb1_ref, rw2t_ref, rb2_ref,
                       out_ref, acc_ref):
    i = pl.program_id(0)

    @pl.when(i == 0)
    def _():
        acc_ref[...] = jnp.zeros_like(acc_ref)

    hn = _layer_update(h_ref, agg_ref, deg_ref, w1_ref, b1_ref, w2_ref,
                       b2_ref)
    base = i * BBH
    valid = (lax.broadcasted_iota(i32, (BBH, 1), 0) + base) < (NNODE // 2)
    hm = jnp.where(valid, hn, 0.0)
    acc_ref[...] += jnp.sum(hm, axis=0, keepdims=True)

    @pl.when(i == NB - 1)
    def _():
        g = acc_ref[...] * (1.0 / NNODE)
        r = jnp.maximum(jnp.dot(g, rw1_ref[...], preferred_element_type=f32)
                        + rb1_ref[...], 0.0)
        o = jnp.sum(r * rw2t_ref[...], axis=1, keepdims=True) + rb2_ref[...]
        out_ref[...] = jax.nn.sigmoid(o)


def _layer_head(h, agg, degp, w1b, b1b, w2b, b2b, rw1s, rb1, rw2t, rb2):
    return pl.pallas_call(
        _layer_head_kernel,
        grid=(NB,),
        in_specs=_LAYER_SPECS + [
            pl.BlockSpec((128, 32), lambda i: (0, 0)),
            pl.BlockSpec((1, 32), lambda i: (0, 0)),
            pl.BlockSpec((1, 32), lambda i: (0, 0)),
            pl.BlockSpec((1, 1), lambda i: (0, 0)),
        ],
        out_specs=pl.BlockSpec((1, 1), lambda i: (0, 0)),
        out_shape=jax.ShapeDtypeStruct((1, 1), f32),
        scratch_shapes=[pltpu.VMEM((1, 128), f32)],
        compiler_params=pltpu.CompilerParams(
            dimension_semantics=("arbitrary",)),
    )(h, agg, degp, w1b, b1b, w2b, b2b, rw1s, rb1, rw2t, rb2)


def _blockdiag(w):
    z = jnp.zeros((128, 128), f32)
    z = z.at[:D, :D].set(w)
    return z.at[D:, D:].set(w)


def kernel(atom_types, coords, edge_index, params):
    del coords
    at2 = jnp.pad(atom_types.astype(i32), (0, P - NNODE)).reshape(PH, 2)
    row = edge_index[0].astype(i32)
    col = edge_index[1].astype(i32)
    rowp = jnp.pad(row, (0, EPAD - NEDGE), constant_values=1 << 20)
    colp = jnp.pad(col, (0, EPAD - NEDGE))
    embp = jnp.pad(params["embedding"].astype(f32), ((0, 128 - VOCAB), (0, 0)))
    embl = jnp.pad(embp, ((0, 0), (0, D)))
    embr = jnp.pad(embp, ((0, 0), (D, 0)))

    colp2 = colp.reshape(EPAD // ECH, ECH)
    h = _embed(at2, embl, embr)
    degp = _degree(rowp).reshape(PH, 2)
    lw = [
        (_blockdiag(layer["W1"].astype(f32)),
         jnp.tile(layer["b1"].astype(f32), 2).reshape(1, 128),
         _blockdiag(layer["W2"].astype(f32)),
         jnp.tile(layer["b2"].astype(f32), 2).reshape(1, 128))
        for layer in params["layers"]
    ]
    for li in range(2):
        agg = _segment_sum(h.reshape(P, D), rowp, colp2).reshape(PH, 128)
        h = _layer(h, agg, degp, *lw[li])
    agg = _segment_sum(h.reshape(P, D), rowp, colp2).reshape(PH, 128)
    rw1s = jnp.concatenate([params["rW1"].astype(f32),
                            params["rW1"].astype(f32)], axis=0)
    out = _layer_head(h, agg, degp, *lw[2],
                      rw1s, params["rb1"].reshape(1, 32),
                      params["rW2"].reshape(1, 32),
                      params["rb2"].reshape(1, 1))
    return out.reshape(1)
